# Initial kernel scaffold; baseline (speedup 1.0000x reference)
import jax
import jax.numpy as jnp
from jax.experimental import pallas as pl


def kernel(init, edge_index_cc, edge_attr_cc, edge_index_cac, edge_attr_cac, edge_index_csc, edge_attr_csc, params):
    raise NotImplementedError("write your pallas kernel here")


# speedup vs baseline: 2.8040x; 2.8040x over previous
import jax
import jax.numpy as jnp
from jax.experimental import pallas as pl

N = 10000
S = 512
D = 128


def _proj_body(init_ref, embs_ref, out_ref):
    init = init_ref[...]
    mask = (init != 0.0).astype(jnp.float32)
    cnt = jnp.sum(mask, axis=1, keepdims=True)
    acc = jnp.dot(mask, embs_ref[...], preferred_element_type=jnp.float32)
    out = acc / jnp.maximum(cnt, 1.0)
    out_ref[...] = jnp.where(cnt > 0, out, 0.0)


def _projection(init, params):
    idxs = jnp.arange(S, dtype=jnp.float32)[:, None]
    h = jax.nn.relu(idxs @ params['proj_W1'].T + params['proj_b1'])
    embs = h @ params['proj_W2'].T + params['proj_b2']
    blk = 1000
    return pl.pallas_call(
        _proj_body,
        grid=(N // blk,),
        in_specs=[
            pl.BlockSpec((blk, S), lambda i: (i, 0)),
            pl.BlockSpec((S, D), lambda i: (0, 0)),
        ],
        out_specs=pl.BlockSpec((blk, D), lambda i: (i, 0)),
        out_shape=jax.ShapeDtypeStruct((N, D), jnp.float32),
    )(init, embs)


def _gcn_conv(x, src, dst, ew, dinv, W, b):
    y = dinv[:, None] * (x @ W.T)
    agg = jnp.zeros((N, D), jnp.float32).at[dst].add(ew[:, None] * y[src])
    return dinv[:, None] * (agg + y) + b


def _gcn_embed(x, ei, ea, params, g):
    src, dst = ei[0], ei[1]
    deg = jnp.ones((N,), jnp.float32).at[dst].add(ea)
    dinv = jax.lax.rsqrt(deg)
    h = x
    for l in range(3):
        h = jax.nn.relu(_gcn_conv(h, src, dst, ea, dinv, params[f'{g}_W{l}'], params[f'{g}_b{l}']))
    return h


def kernel(init, edge_index_cc, edge_attr_cc, edge_index_cac, edge_attr_cac,
           edge_index_csc, edge_attr_csc, params):
    x0 = _projection(init, params)
    outs = []
    for g, ei, ea in (('cc', edge_index_cc, edge_attr_cc),
                      ('cac', edge_index_cac, edge_attr_cac),
                      ('csc', edge_index_csc, edge_attr_csc)):
        outs.append(_gcn_embed(x0, ei, ea, params, g))
    combined = jnp.stack(outs, axis=0)
    scores = jnp.concatenate(
        [combined[i] @ params['att_W'].T + params['att_b'] for i in range(3)], axis=1)
    attw = jax.nn.softmax(scores, axis=1)
    weighted = jnp.zeros_like(combined[0])
    for i in range(3):
        weighted = weighted + attw[:, i][:, None] * combined[i]
    return weighted


# speedup vs baseline: 7.4833x; 2.6688x over previous
import dataclasses
import functools

import jax
import jax.numpy as jnp
from jax import lax
from jax.experimental import pallas as pl
from jax.experimental.pallas import tpu as pltpu
from jax.experimental.pallas import tpu_sc as plsc

N = 10000
S = 512
D = 128
E = 320000

GSZ = 128
GROUPS = 79
EPT = GROUPS * GSZ
E_PAD = 32 * EPT
DEG_PAD = 10240
N_PAD = 10240
ROWS_PT = N_PAD // 16
RBLK = 128

_vmesh = plsc.VectorSubcoreMesh(core_axis_name="c", subcore_axis_name="s")

_sc_params = pltpu.CompilerParams()
if "needs_layout_passes" in pltpu.CompilerParams.__dataclass_fields__:
    _sc_params = dataclasses.replace(_sc_params, needs_layout_passes=False)


def _deg_call(dstp, wp):
    @functools.partial(
        pl.kernel,
        out_type=jax.ShapeDtypeStruct((2, DEG_PAD), jnp.float32),
        mesh=_vmesh,
        compiler_params=_sc_params,
        scratch_types=[
            pltpu.VMEM((GROUPS, GSZ), jnp.int32),
            pltpu.VMEM((GROUPS, GSZ), jnp.float32),
            pltpu.VMEM((DEG_PAD,), jnp.float32),
            pltpu.VMEM((640,), jnp.float32),
            pltpu.VMEM_SHARED((16, DEG_PAD), jnp.float32),
        ],
    )
    def k(dst_hbm, w_hbm, out_hbm, dst_v, w_v, deg_v, tmp_v, sh):
        c = lax.axis_index("c")
        s = lax.axis_index("s")
        wid = c * 16 + s
        pltpu.sync_copy(dst_hbm.at[wid], dst_v)
        pltpu.sync_copy(w_hbm.at[wid], w_v)

        @pl.loop(0, DEG_PAD // 16)
        def _(i):
            deg_v[pl.ds(i * 16, 16)] = jnp.zeros((16,), jnp.float32)

        @pl.loop(0, GROUPS)
        def _(g):
            for kk in range(8):
                d16 = dst_v[g, pl.ds(kk * 16, 16)]
                w16 = w_v[g, pl.ds(kk * 16, 16)]
                plsc.addupdate_scatter(deg_v, [d16], w16)

        pltpu.sync_copy(deg_v, sh.at[s])
        plsc.subcore_barrier()

        @pl.loop(0, 40)
        def _(i):
            deg_v[pl.ds(i * 16, 16)] = jnp.zeros((16,), jnp.float32)

        for t in range(16):
            pltpu.sync_copy(sh.at[t, pl.ds(s * 640, 640)], tmp_v)

            @pl.loop(0, 40)
            def _(i):
                deg_v[pl.ds(i * 16, 16)] = (
                    deg_v[pl.ds(i * 16, 16)] + tmp_v[pl.ds(i * 16, 16)]
                )

        pltpu.sync_copy(deg_v.at[pl.ds(0, 640)], out_hbm.at[c, pl.ds(s * 640, 640)])

    return k(dstp, wp)


def _agg_call(y, srcp, dstp, wp):
    @functools.partial(
        pl.kernel,
        out_type=jax.ShapeDtypeStruct((2, N_PAD, D), jnp.float32),
        mesh=_vmesh,
        compiler_params=_sc_params,
        scratch_types=[
            pltpu.VMEM((GROUPS, GSZ), jnp.int32),
            pltpu.VMEM((GROUPS, GSZ), jnp.int32),
            pltpu.VMEM((GROUPS, GSZ), jnp.float32),
            pltpu.VMEM((GSZ, D), jnp.float32),
            pltpu.VMEM_SHARED((N_PAD, D), jnp.float32),
            pltpu.SemaphoreType.DMA,
        ],
    )
    def k(y_hbm, src_hbm, dst_hbm, w_hbm, out_hbm,
          src_v, dst_v, w_v, rows_v, acc_sh, sem):
        c = lax.axis_index("c")
        s = lax.axis_index("s")
        wid = c * 16 + s

        @pl.loop(0, RBLK)
        def _(r):
            for kk in range(8):
                rows_v[r, pl.ds(kk * 16, 16)] = jnp.zeros((16,), jnp.float32)

        for blk in range(5):
            pltpu.sync_copy(rows_v, acc_sh.at[pl.ds(s * ROWS_PT + blk * RBLK, RBLK)])
        plsc.subcore_barrier()

        pltpu.sync_copy(src_hbm.at[wid], src_v)
        pltpu.sync_copy(dst_hbm.at[wid], dst_v)
        pltpu.sync_copy(w_hbm.at[wid], w_v)

        @pl.loop(0, GROUPS)
        def _(g):
            pltpu.async_copy(y_hbm.at[src_v.at[g]], rows_v, sem).wait()

            @pl.loop(0, GSZ // 16)
            def _(e16):
                w16 = w_v[g, pl.ds(e16 * 16, 16)]
                for i in range(16):
                    ws = w16[i]
                    e = e16 * 16 + i
                    for kk in range(8):
                        rows_v[e, pl.ds(kk * 16, 16)] = (
                            rows_v[e, pl.ds(kk * 16, 16)] * ws)

            pltpu.sync_copy(rows_v, acc_sh.at[dst_v.at[g]], add=True)

        plsc.subcore_barrier()
        for blk in range(5):
            base = s * ROWS_PT + blk * RBLK
            pltpu.sync_copy(acc_sh.at[pl.ds(base, RBLK)], rows_v)
            pltpu.sync_copy(rows_v, out_hbm.at[c, pl.ds(base, RBLK)])

    return k(y, srcp, dstp, wp)


_BLK = 1000


def _proj_body(init_ref, embs_ref, out_ref):
    init = init_ref[...]
    mask = (init != 0.0).astype(jnp.float32)
    cnt = jnp.sum(mask, axis=1, keepdims=True)
    acc = jnp.dot(mask, embs_ref[...], preferred_element_type=jnp.float32)
    out = acc / jnp.maximum(cnt, 1.0)
    out_ref[...] = jnp.where(cnt > 0, out, 0.0)


def _projection(init, params):
    idxs = jnp.arange(S, dtype=jnp.float32)[:, None]
    h = jax.nn.relu(idxs @ params['proj_W1'].T + params['proj_b1'])
    embs = h @ params['proj_W2'].T + params['proj_b2']
    return pl.pallas_call(
        _proj_body,
        grid=(N // _BLK,),
        in_specs=[
            pl.BlockSpec((_BLK, S), lambda i: (i, 0)),
            pl.BlockSpec((S, D), lambda i: (0, 0)),
        ],
        out_specs=pl.BlockSpec((_BLK, D), lambda i: (i, 0)),
        out_shape=jax.ShapeDtypeStruct((N, D), jnp.float32),
    )(init, embs)


def _scale_matmul_body(x_ref, dinv_ref, wt_ref, out_ref):
    out_ref[...] = dinv_ref[...] * jnp.dot(
        x_ref[...], wt_ref[...], preferred_element_type=jnp.float32)


def _scale_matmul(x, dinv, wt):
    return pl.pallas_call(
        _scale_matmul_body,
        grid=(N // _BLK,),
        in_specs=[
            pl.BlockSpec((_BLK, D), lambda i: (i, 0)),
            pl.BlockSpec((_BLK, 1), lambda i: (i, 0)),
            pl.BlockSpec((D, D), lambda i: (0, 0)),
        ],
        out_specs=pl.BlockSpec((_BLK, D), lambda i: (i, 0)),
        out_shape=jax.ShapeDtypeStruct((N, D), jnp.float32),
    )(x, dinv, wt)


def _layer_body(p0_ref, p1_ref, y_ref, dinv_ref, b_ref, wt_ref, out_ref):
    dinv = dinv_ref[...]
    h = jax.nn.relu(dinv * (p0_ref[...] + p1_ref[...] + y_ref[...]) + b_ref[...])
    out_ref[...] = dinv * jnp.dot(h, wt_ref[...], preferred_element_type=jnp.float32)


def _layer(parts, y, dinv, b, wt_next):
    return pl.pallas_call(
        _layer_body,
        grid=(N // _BLK,),
        in_specs=[
            pl.BlockSpec((_BLK, D), lambda i: (i, 0)),
            pl.BlockSpec((_BLK, D), lambda i: (i, 0)),
            pl.BlockSpec((_BLK, D), lambda i: (i, 0)),
            pl.BlockSpec((_BLK, 1), lambda i: (i, 0)),
            pl.BlockSpec((1, D), lambda i: (0, 0)),
            pl.BlockSpec((D, D), lambda i: (0, 0)),
        ],
        out_specs=pl.BlockSpec((_BLK, D), lambda i: (i, 0)),
        out_shape=jax.ShapeDtypeStruct((N, D), jnp.float32),
    )(parts[0], parts[1], y, dinv, b, wt_next)


def _layer_last_body(p0_ref, p1_ref, y_ref, dinv_ref, b_ref, out_ref):
    dinv = dinv_ref[...]
    out_ref[...] = jax.nn.relu(
        dinv * (p0_ref[...] + p1_ref[...] + y_ref[...]) + b_ref[...])


def _layer_last(parts, y, dinv, b):
    return pl.pallas_call(
        _layer_last_body,
        grid=(N // _BLK,),
        in_specs=[
            pl.BlockSpec((_BLK, D), lambda i: (i, 0)),
            pl.BlockSpec((_BLK, D), lambda i: (i, 0)),
            pl.BlockSpec((_BLK, D), lambda i: (i, 0)),
            pl.BlockSpec((_BLK, 1), lambda i: (i, 0)),
            pl.BlockSpec((1, D), lambda i: (0, 0)),
        ],
        out_specs=pl.BlockSpec((_BLK, D), lambda i: (i, 0)),
        out_shape=jax.ShapeDtypeStruct((N, D), jnp.float32),
    )(parts[0], parts[1], y, dinv, b)


def _att_body(h0_ref, h1_ref, h2_ref, aw_ref, ab_ref, out_ref):
    aw = aw_ref[...]
    ab = ab_ref[...]
    h0, h1, h2 = h0_ref[...], h1_ref[...], h2_ref[...]
    s0 = jnp.sum(h0 * aw, axis=1, keepdims=True) + ab
    s1 = jnp.sum(h1 * aw, axis=1, keepdims=True) + ab
    s2 = jnp.sum(h2 * aw, axis=1, keepdims=True) + ab
    m = jnp.maximum(jnp.maximum(s0, s1), s2)
    e0 = jnp.exp(s0 - m)
    e1 = jnp.exp(s1 - m)
    e2 = jnp.exp(s2 - m)
    z = e0 + e1 + e2
    out_ref[...] = (e0 * h0 + e1 * h1 + e2 * h2) / z


def _attention(hs, att_w, att_b):
    return pl.pallas_call(
        _att_body,
        grid=(N // _BLK,),
        in_specs=[
            pl.BlockSpec((_BLK, D), lambda i: (i, 0)),
            pl.BlockSpec((_BLK, D), lambda i: (i, 0)),
            pl.BlockSpec((_BLK, D), lambda i: (i, 0)),
            pl.BlockSpec((1, D), lambda i: (0, 0)),
            pl.BlockSpec((1, 1), lambda i: (0, 0)),
        ],
        out_specs=pl.BlockSpec((_BLK, D), lambda i: (i, 0)),
        out_shape=jax.ShapeDtypeStruct((N, D), jnp.float32),
    )(hs[0], hs[1], hs[2], att_w, att_b)


def _pad_edges(ei, ea):
    src = ei[0].astype(jnp.int32)
    dst = ei[1].astype(jnp.int32)
    pad = E_PAD - E
    srcp = jnp.concatenate([src, jnp.zeros((pad,), jnp.int32)]).reshape(32, GROUPS, GSZ)
    dstp = jnp.concatenate([dst, jnp.zeros((pad,), jnp.int32)]).reshape(32, GROUPS, GSZ)
    wp = jnp.concatenate([ea, jnp.zeros((pad,), jnp.float32)]).reshape(32, GROUPS, GSZ)
    return srcp, dstp, wp


def kernel(init, edge_index_cc, edge_attr_cc, edge_index_cac, edge_attr_cac,
           edge_index_csc, edge_attr_csc, params):
    p = params
    x0 = _projection(init, p)
    hs = []
    for g, ei, ea in (('cc', edge_index_cc, edge_attr_cc),
                      ('cac', edge_index_cac, edge_attr_cac),
                      ('csc', edge_index_csc, edge_attr_csc)):
        srcp, dstp, wp = _pad_edges(ei, ea)
        degp = _deg_call(dstp, wp)
        dinv = lax.rsqrt(1.0 + degp[0, :N] + degp[1, :N])[:, None]
        y = _scale_matmul(x0, dinv, p[f'{g}_W0'].T)
        for l in range(3):
            parts = _agg_call(y, srcp, dstp, wp)
            b = p[f'{g}_b{l}'][None, :]
            if l < 2:
                y = _layer(parts, y, dinv, b, p[f'{g}_W{l + 1}'].T)
            else:
                hs.append(_layer_last(parts, y, dinv, b))
    return _attention(hs, p['att_W'], p['att_b'][None, :][:, :1])
